# baseline (speedup 1.0000x reference)
import functools

import jax
import jax.numpy as jnp
from jax import lax
from jax.experimental import pallas as pl
from jax.experimental.pallas import tpu as pltpu
from jax.experimental.pallas import tpu_sc as plsc

N, C, H, W = 4, 96, 384, 384
P = N * H * W
NWORK = 32
PPW = P // NWORK
K = 128
NCHUNK = PPW // K
HW = H * W
L = 16


def _sc_grid_sample(table, gx, gy):
    mesh = plsc.VectorSubcoreMesh(core_axis_name="c", subcore_axis_name="s")

    @functools.partial(
        pl.kernel,
        mesh=mesh,
        out_type=jax.ShapeDtypeStruct((P, C), jnp.float32),
        compiler_params=pltpu.CompilerParams(
            needs_layout_passes=False, use_tc_tiling_on_sc=False),
        scratch_types=[
            pltpu.VMEM((K,), jnp.float32),
            pltpu.VMEM((K,), jnp.float32),
            pltpu.VMEM((K,), jnp.int32),
            pltpu.VMEM((K,), jnp.int32),
            pltpu.VMEM((K,), jnp.int32),
            pltpu.VMEM((K,), jnp.int32),
            pltpu.VMEM((K,), jnp.float32),
            pltpu.VMEM((K,), jnp.float32),
            pltpu.VMEM((K,), jnp.float32),
            pltpu.VMEM((K,), jnp.float32),
            pltpu.VMEM((K, C), jnp.float32),
            pltpu.VMEM((K, C), jnp.float32),
            pltpu.VMEM((K, C), jnp.float32),
            pltpu.VMEM((K, C), jnp.float32),
            pltpu.VMEM((K, C), jnp.float32),
            pltpu.SemaphoreType.DMA,
        ],
    )
    def k(table_hbm, gx_hbm, gy_hbm, out_hbm,
          gx_v, gy_v, i00, i01, i10, i11, w00, w01, w10, w11,
          r00, r01, r10, r11, out_v, sem):
        wid = lax.axis_index("s") * 2 + lax.axis_index("c")
        nbase = (wid // 8) * HW

        lanes = lax.iota(jnp.int32, L)

        def chunk_body(t, carry):
            base = wid * PPW + t * K
            pltpu.sync_copy(gx_hbm.at[pl.ds(base, K)], gx_v)
            pltpu.sync_copy(gy_hbm.at[pl.ds(base, K)], gy_v)

            for j in range(K // L):
                sl = pl.ds(j * L, L)
                x = gx_v[sl]
                y = gy_v[sl]
                ix = ((x + 1.0) * jnp.float32(W) - 1.0) * 0.5
                iy = ((y + 1.0) * jnp.float32(H) - 1.0) * 0.5
                x0t = ix.astype(jnp.int32).astype(jnp.float32)
                x0f = jnp.where(x0t > ix, x0t - 1.0, x0t)
                y0t = iy.astype(jnp.int32).astype(jnp.float32)
                y0f = jnp.where(y0t > iy, y0t - 1.0, y0t)
                fx = ix - x0f
                fy = iy - y0f
                x0 = x0f.astype(jnp.int32)
                y0 = y0f.astype(jnp.int32)
                x1 = x0 + 1
                y1 = y0 + 1
                zero = jnp.zeros((L,), jnp.float32)
                vx0 = jnp.where((x0 >= 0) & (x0 <= W - 1), 1.0 - fx, zero)
                vx1 = jnp.where((x1 >= 0) & (x1 <= W - 1), fx, zero)
                vy0 = jnp.where((y0 >= 0) & (y0 <= H - 1), 1.0 - fy, zero)
                vy1 = jnp.where((y1 >= 0) & (y1 <= H - 1), fy, zero)
                x0c = jnp.minimum(jnp.maximum(x0, 0), W - 1)
                x1c = jnp.minimum(jnp.maximum(x1, 0), W - 1)
                y0c = jnp.minimum(jnp.maximum(y0, 0), H - 1)
                y1c = jnp.minimum(jnp.maximum(y1, 0), H - 1)
                r0 = nbase + y0c * W
                r1 = nbase + y1c * W
                i00[sl] = r0 + x0c
                i01[sl] = r0 + x1c
                i10[sl] = r1 + x0c
                i11[sl] = r1 + x1c
                w00[sl] = vy0 * vx0
                w01[sl] = vy0 * vx1
                w10[sl] = vy1 * vx0
                w11[sl] = vy1 * vx1

            c0 = pltpu.async_copy(table_hbm.at[i00], r00, sem)
            c1 = pltpu.async_copy(table_hbm.at[i01], r01, sem)
            c2 = pltpu.async_copy(table_hbm.at[i10], r10, sem)
            c3 = pltpu.async_copy(table_hbm.at[i11], r11, sem)
            c0.wait()
            c1.wait()
            c2.wait()
            c3.wait()

            def pix_body(kk, carry2):
                bidx = jnp.full((L,), kk, jnp.int32)
                a00 = plsc.load_gather(w00, [bidx])
                a01 = plsc.load_gather(w01, [bidx])
                a10 = plsc.load_gather(w10, [bidx])
                a11 = plsc.load_gather(w11, [bidx])
                for j in range(C // L):
                    col = lanes + (j * L)
                    v00 = plsc.load_gather(r00, [bidx, col])
                    v01 = plsc.load_gather(r01, [bidx, col])
                    v10 = plsc.load_gather(r10, [bidx, col])
                    v11 = plsc.load_gather(r11, [bidx, col])
                    acc = v00 * a00 + v01 * a01 + v10 * a10 + v11 * a11
                    plsc.store_scatter(out_v, [bidx, col], acc)
                return carry2

            lax.fori_loop(0, K, pix_body, 0)

            pltpu.sync_copy(out_v, out_hbm.at[pl.ds(base, K)])
            return carry

        lax.fori_loop(0, NCHUNK, chunk_body, 0)

    return k(table, gx, gy)


def kernel(input, grid):
    table = input.transpose(0, 2, 3, 1).reshape(N * H * W, C)
    gx = grid[..., 0].reshape(P)
    gy = grid[..., 1].reshape(P)
    out_rows = _sc_grid_sample(table, gx, gy)
    return out_rows.reshape(N, H, W, C).transpose(0, 3, 1, 2)


# speedup vs baseline: 1.1510x; 1.1510x over previous
import functools

import jax
import jax.numpy as jnp
from jax import lax
from jax.experimental import pallas as pl
from jax.experimental.pallas import tpu as pltpu
from jax.experimental.pallas import tpu_sc as plsc

N, C, H, W = 4, 96, 384, 384
P = N * H * W
NWORK = 32
PPW = P // NWORK
K = 128
NCHUNK = PPW // K
HW = H * W
L = 16


def _sc_grid_sample(table, gx, gy):
    mesh = plsc.VectorSubcoreMesh(core_axis_name="c", subcore_axis_name="s")

    @functools.partial(
        pl.kernel,
        mesh=mesh,
        out_type=jax.ShapeDtypeStruct((P, C), jnp.bfloat16),
        compiler_params=pltpu.CompilerParams(
            needs_layout_passes=False, use_tc_tiling_on_sc=False),
        scratch_types=[
            pltpu.VMEM((K,), jnp.float32),
            pltpu.VMEM((K,), jnp.float32),
        ] + [
            pltpu.VMEM((K,), jnp.int32)
            for _ in range(8)
        ] + [
            pltpu.VMEM((K,), jnp.float32)
            for _ in range(8)
        ] + [
            pltpu.VMEM((K, C), jnp.bfloat16)
            for _ in range(8)
        ] + [
            pltpu.VMEM((K, C), jnp.bfloat16)
            for _ in range(2)
        ] + [
            pltpu.SemaphoreType.DMA,
            pltpu.SemaphoreType.DMA,
        ],
    )
    def k(table_hbm, gx_hbm, gy_hbm, out_hbm, gx_v, gy_v, *sc):
        idx = (sc[0:4], sc[4:8])
        wts = (sc[8:12], sc[12:16])
        rws = (sc[16:20], sc[20:24])
        out_v = sc[24:26]
        gsem = sc[26:28]

        wid = lax.axis_index("s") * 2 + lax.axis_index("c")
        nbase = (wid // 8) * HW

        def stage(t, b):
            base = wid * PPW + t * K
            pltpu.sync_copy(gx_hbm.at[pl.ds(base, K)], gx_v)
            pltpu.sync_copy(gy_hbm.at[pl.ds(base, K)], gy_v)
            i00, i01, i10, i11 = idx[b]
            w00, w01, w10, w11 = wts[b]
            for j in range(K // L):
                sl = pl.ds(j * L, L)
                x = gx_v[sl]
                y = gy_v[sl]
                ix = ((x + 1.0) * jnp.float32(W) - 1.0) * 0.5
                iy = ((y + 1.0) * jnp.float32(H) - 1.0) * 0.5
                x0t = ix.astype(jnp.int32).astype(jnp.float32)
                x0f = jnp.where(x0t > ix, x0t - 1.0, x0t)
                y0t = iy.astype(jnp.int32).astype(jnp.float32)
                y0f = jnp.where(y0t > iy, y0t - 1.0, y0t)
                fx = ix - x0f
                fy = iy - y0f
                x0 = x0f.astype(jnp.int32)
                y0 = y0f.astype(jnp.int32)
                x1 = x0 + 1
                y1 = y0 + 1
                zero = jnp.zeros((L,), jnp.float32)
                vx0 = jnp.where((x0 >= 0) & (x0 <= W - 1), 1.0 - fx, zero)
                vx1 = jnp.where((x1 >= 0) & (x1 <= W - 1), fx, zero)
                vy0 = jnp.where((y0 >= 0) & (y0 <= H - 1), 1.0 - fy, zero)
                vy1 = jnp.where((y1 >= 0) & (y1 <= H - 1), fy, zero)
                x0c = jnp.minimum(jnp.maximum(x0, 0), W - 1)
                x1c = jnp.minimum(jnp.maximum(x1, 0), W - 1)
                y0c = jnp.minimum(jnp.maximum(y0, 0), H - 1)
                y1c = jnp.minimum(jnp.maximum(y1, 0), H - 1)
                r0 = nbase + y0c * W
                r1 = nbase + y1c * W
                i00[sl] = r0 + x0c
                i01[sl] = r0 + x1c
                i10[sl] = r1 + x0c
                i11[sl] = r1 + x1c
                w00[sl] = vy0 * vx0
                w01[sl] = vy0 * vx1
                w10[sl] = vy1 * vx0
                w11[sl] = vy1 * vx1
            r00, r01, r10, r11 = rws[b]
            return (pltpu.async_copy(table_hbm.at[i00], r00, gsem[b]),
                    pltpu.async_copy(table_hbm.at[i01], r01, gsem[b]),
                    pltpu.async_copy(table_hbm.at[i10], r10, gsem[b]),
                    pltpu.async_copy(table_hbm.at[i11], r11, gsem[b]))

        def blend(t, b):
            base = wid * PPW + t * K
            w00, w01, w10, w11 = wts[b]
            r00, r01, r10, r11 = rws[b]
            ov = out_v[b]
            pltpu.make_async_copy(table_hbm.at[idx[b][0]], r00, gsem[b]).wait()
            pltpu.make_async_copy(table_hbm.at[idx[b][1]], r01, gsem[b]).wait()
            pltpu.make_async_copy(table_hbm.at[idx[b][2]], r10, gsem[b]).wait()
            pltpu.make_async_copy(table_hbm.at[idx[b][3]], r11, gsem[b]).wait()

            def pix_body(kk, carry):
                bidx = jnp.full((L,), kk, jnp.int32)
                a00 = plsc.load_gather(w00, [bidx])
                a01 = plsc.load_gather(w01, [bidx])
                a10 = plsc.load_gather(w10, [bidx])
                a11 = plsc.load_gather(w11, [bidx])
                b00 = plsc.pack(a00, a00, format=plsc.PackFormat.INTERLEAVED)
                b01 = plsc.pack(a01, a01, format=plsc.PackFormat.INTERLEAVED)
                b10 = plsc.pack(a10, a10, format=plsc.PackFormat.INTERLEAVED)
                b11 = plsc.pack(a11, a11, format=plsc.PackFormat.INTERLEAVED)
                for j in range(C // (2 * L)):
                    sl = pl.ds(j * 2 * L, 2 * L)
                    acc = (r00[kk, sl] * b00 + r01[kk, sl] * b01
                           + r10[kk, sl] * b10 + r11[kk, sl] * b11)
                    ov[kk, sl] = acc
                return carry

            lax.fori_loop(0, K, pix_body, 0)
            pltpu.sync_copy(ov, out_hbm.at[pl.ds(base, K)])

        stage(0, 0)

        def chunk_pair(g, carry):
            t = g * 2
            stage(t + 1, 1)
            blend(t, 0)
            stage(t + 2, 0)
            blend(t + 1, 1)
            return carry

        lax.fori_loop(0, NCHUNK // 2, chunk_pair, 0)
        for q in range(4):
            pltpu.make_async_copy(
                table_hbm.at[idx[0][q]], rws[0][q], gsem[0]).wait()

    return k(table, gx, gy)


def kernel(input, grid):
    table = input.transpose(0, 2, 3, 1).reshape(P, C).astype(jnp.bfloat16)
    pad = jnp.zeros((K,), jnp.float32)
    gx = jnp.concatenate([grid[..., 0].reshape(P), pad])
    gy = jnp.concatenate([grid[..., 1].reshape(P), pad])
    out_rows = _sc_grid_sample(table, gx, gy)
    return (out_rows.astype(jnp.float32)
            .reshape(N, H, W, C).transpose(0, 3, 1, 2))
